# baseline (speedup 1.0000x reference)
import jax
import jax.numpy as jnp
from jax import lax
from jax.experimental import pallas as pl
from jax.experimental.pallas import tpu as pltpu
from jax.experimental.pallas import tpu_sc as plsc

N = 10000
E = 320000
D = 128

NC = 2
NS = 16
CHUNK = 128
NCHUNK = 2560
E_PAD = NCHUNK * CHUNK
CPW_ALL = NCHUNK // NS
CPW0 = 160
CPW1 = 0
N_TAB = 10112
ZROWS = N_TAB // NS
GRP = 8

_MESH = plsc.VectorSubcoreMesh(
    core_axis_name="c", subcore_axis_name="s", num_cores=NC, num_subcores=NS)


def _sc_builder(mode):
  out_type = jax.ShapeDtypeStruct((NC, N_TAB, D), jnp.float32)
  scratch = [
      pltpu.VMEM((GRP, CHUNK), jnp.int32),
      pltpu.VMEM((GRP, CHUNK), jnp.int32),
      pltpu.VMEM((CHUNK, D), jnp.float32),
      pltpu.VMEM((CHUNK, D), jnp.float32),
      pltpu.VMEM_SHARED((N_TAB, D), jnp.float32),
      pltpu.SemaphoreType.DMA,
      pltpu.SemaphoreType.DMA,
  ]

  def body(table_hbm, ones_hbm, src_hbm, dst_hbm, znd_hbm, p_hbm,
           srcv, dstv, rows_a, rows_b, agg_s, sem_a, sem_b):
    c = lax.axis_index("c")
    s = lax.axis_index("s")
    bufs = [rows_a, rows_b]
    sems = [sem_a, sem_b]

    pltpu.sync_copy(znd_hbm.at[pl.ds(s * ZROWS, ZROWS)],
                    agg_s.at[pl.ds(s * ZROWS, ZROWS)])
    if mode == "l1":
      @pl.when(c == 1)
      def _():
        pltpu.sync_copy(ones_hbm, rows_a)
    plsc.subcore_barrier()

    def agg_group(start):
      def group(g, carry):
        base = start + g * GRP
        pltpu.sync_copy(src_hbm.at[pl.ds(base, GRP)], srcv)
        pltpu.sync_copy(dst_hbm.at[pl.ds(base, GRP)], dstv)
        descs = [None, None]
        descs[0] = pltpu.async_copy(table_hbm.at[srcv.at[0]], bufs[0], sems[0])
        for j in range(GRP):
          if j + 1 < GRP:
            descs[(j + 1) % 2] = pltpu.async_copy(
                table_hbm.at[srcv.at[j + 1]], bufs[(j + 1) % 2],
                sems[(j + 1) % 2])
          descs[j % 2].wait()
          pltpu.sync_copy(bufs[j % 2], agg_s.at[dstv.at[j]], add=True)
        return carry
      return group

    def cnt_group(start):
      def group(g, carry):
        base = start + g * GRP
        pltpu.sync_copy(dst_hbm.at[pl.ds(base, GRP)], dstv)
        for j in range(GRP):
          pltpu.sync_copy(rows_a, agg_s.at[dstv.at[j]], add=True)
        return carry
      return group

    if mode == "l1":
      @pl.when(c == 0)
      def _():
        lax.fori_loop(0, CPW_ALL // GRP, agg_group(s * CPW_ALL), 0)

      @pl.when(c == 1)
      def _():
        lax.fori_loop(0, CPW_ALL // GRP, cnt_group(s * CPW_ALL), 0)
    else:
      @pl.when(c == 0)
      def _():
        lax.fori_loop(0, CPW0 // GRP, agg_group(s * CPW0), 0)

      if CPW1 > 0:
        @pl.when(c == 1)
        def _():
          lax.fori_loop(0, CPW1 // GRP, agg_group(NS * CPW0 + s * CPW1), 0)

    plsc.subcore_barrier()
    pltpu.sync_copy(agg_s.at[pl.ds(s * ZROWS, ZROWS)],
                    p_hbm.at[c, pl.ds(s * ZROWS, ZROWS)])

  return pl.kernel(body, out_type=out_type, mesh=_MESH, scratch_types=scratch)


_sc_l1 = _sc_builder("l1")
_sc_l2 = _sc_builder("l2")


def _tc1_body(pc_ref, x_ref, wl_ref, b_ref, wr_ref, o_ref):
  cnt = jnp.maximum(pc_ref[1, :, 0:1], 1.0)
  agg = pc_ref[0] / cnt
  o_ref[...] = jnp.maximum(
      jnp.dot(agg, wl_ref[...], preferred_element_type=jnp.float32)
      + b_ref[...]
      + jnp.dot(x_ref[...], wr_ref[...], preferred_element_type=jnp.float32),
      0.0)


def _tc2_body(p2_ref, cnt_ref, x_ref, wl_ref, b_ref, wr_ref, o_ref):
  cnt = jnp.maximum(cnt_ref[0, :, 0:1], 1.0)
  agg = (p2_ref[0] + p2_ref[1]) / cnt
  o_ref[...] = jnp.maximum(
      jnp.dot(agg, wl_ref[...], preferred_element_type=jnp.float32)
      + b_ref[...]
      + jnp.dot(x_ref[...], wr_ref[...], preferred_element_type=jnp.float32),
      0.0)


_R = 1000
_COMMON_SPECS = [
    pl.BlockSpec((_R, D), lambda i: (i, 0)),
    pl.BlockSpec((D, D), lambda i: (0, 0)),
    pl.BlockSpec((1, D), lambda i: (0, 0)),
    pl.BlockSpec((D, D), lambda i: (0, 0)),
]
_OUT_SPEC = pl.BlockSpec((_R, D), lambda i: (i, 0))


def _tc_layer1(pc, x, wlT, b, wrT):
  return pl.pallas_call(
      _tc1_body,
      grid=(N // _R,),
      in_specs=[pl.BlockSpec((NC, _R, D), lambda i: (0, i, 0))] + _COMMON_SPECS,
      out_specs=_OUT_SPEC,
      out_shape=jax.ShapeDtypeStruct((N, D), jnp.float32),
  )(pc, x, wlT, b.reshape(1, D), wrT)


def _tc_layer2(p2, pc, x, wlT, b, wrT):
  return pl.pallas_call(
      _tc2_body,
      grid=(N // _R,),
      in_specs=[pl.BlockSpec((NC, _R, D), lambda i: (0, i, 0)),
                pl.BlockSpec((1, _R, D), lambda i: (1, i, 0))] + _COMMON_SPECS,
      out_specs=_OUT_SPEC,
      out_shape=jax.ShapeDtypeStruct((N, D), jnp.float32),
  )(p2, pc, x, wlT, b.reshape(1, D), wrT)


def kernel(x, edge_index, W1l, b1, W1r, W2l, b2, W2r):
  src = edge_index[0].astype(jnp.int32)
  dst = edge_index[1].astype(jnp.int32)
  pad = E_PAD - E
  pad_dst = N + (jnp.arange(pad, dtype=jnp.int32) % (N_TAB - N))
  src2 = jnp.concatenate([src, jnp.zeros((pad,), jnp.int32)]).reshape(-1, CHUNK)
  dst2 = jnp.concatenate([dst, pad_dst]).reshape(-1, CHUNK)

  znd = jnp.zeros((N_TAB, D), jnp.float32)
  ones = jnp.ones((CHUNK, D), jnp.float32)

  pc = _sc_l1(x, ones, src2, dst2, znd)
  h = _tc_layer1(pc, x, W1l.T, b1, W1r.T)
  p2 = _sc_l2(h, ones, src2, dst2, znd)
  out = _tc_layer2(p2, pc, h, W2l.T, b2, W2r.T)
  return out


# speedup vs baseline: 1.0418x; 1.0418x over previous
import jax
import jax.numpy as jnp
from jax import lax
from jax.experimental import pallas as pl
from jax.experimental.pallas import tpu as pltpu
from jax.experimental.pallas import tpu_sc as plsc

N = 10000
E = 320000
D = 128

NC = 2
NS = 16
CHUNK = 128
NCHUNK = 2560
E_PAD = NCHUNK * CHUNK
CPW = NCHUNK // (NC * NS)
N_TAB = 10112
ZROWS = N_TAB // NS
GRP = 8

_MESH = plsc.VectorSubcoreMesh(
    core_axis_name="c", subcore_axis_name="s", num_cores=NC, num_subcores=NS)


def _sc_builder(kind):
  out_type = jax.ShapeDtypeStruct((NC, N_TAB, D), jnp.float32)
  scratch = [
      pltpu.VMEM((GRP, CHUNK), jnp.int32),
      pltpu.VMEM((GRP, CHUNK), jnp.int32),
      pltpu.VMEM((CHUNK, D), jnp.float32),
      pltpu.VMEM((CHUNK, D), jnp.float32),
      pltpu.VMEM_SHARED((N_TAB, D), jnp.float32),
      pltpu.SemaphoreType.DMA,
      pltpu.SemaphoreType.DMA,
  ]

  def body(table_hbm, src_hbm, dst_hbm, znd_hbm, p_hbm,
           srcv, dstv, rows_a, rows_b, agg_s, sem_a, sem_b):
    c = lax.axis_index("c")
    s = lax.axis_index("s")
    bufs = [rows_a, rows_b]
    sems = [sem_a, sem_b]

    pltpu.sync_copy(znd_hbm.at[pl.ds(s * ZROWS, ZROWS)],
                    agg_s.at[pl.ds(s * ZROWS, ZROWS)])
    if kind == "cnt":
      pltpu.sync_copy(table_hbm, rows_a)
    plsc.subcore_barrier()

    start = (c * NS + s) * CPW

    def group(g, carry):
      base = start + g * GRP
      if kind == "agg":
        pltpu.sync_copy(src_hbm.at[pl.ds(base, GRP)], srcv)
      pltpu.sync_copy(dst_hbm.at[pl.ds(base, GRP)], dstv)

      if kind == "agg":
        descs = [None, None]
        descs[0] = pltpu.async_copy(table_hbm.at[srcv.at[0]], bufs[0], sems[0])
        for j in range(GRP):
          if j + 1 < GRP:
            descs[(j + 1) % 2] = pltpu.async_copy(
                table_hbm.at[srcv.at[j + 1]], bufs[(j + 1) % 2],
                sems[(j + 1) % 2])
          descs[j % 2].wait()
          pltpu.sync_copy(bufs[j % 2], agg_s.at[dstv.at[j]], add=True)
      else:
        for j in range(GRP):
          pltpu.sync_copy(rows_a, agg_s.at[dstv.at[j]], add=True)
      return carry

    lax.fori_loop(0, CPW // GRP, group, 0)
    plsc.subcore_barrier()
    pltpu.sync_copy(agg_s.at[pl.ds(s * ZROWS, ZROWS)],
                    p_hbm.at[c, pl.ds(s * ZROWS, ZROWS)])

  return pl.kernel(body, out_type=out_type, mesh=_MESH, scratch_types=scratch)


_sc_agg = _sc_builder("agg")
_sc_cnt = _sc_builder("cnt")


def _tc_body(p_ref, c_ref, x_ref, wl_ref, b_ref, wr_ref, o_ref):
  cnt = jnp.maximum(c_ref[0, :, 0:1] + c_ref[1, :, 0:1], 1.0)
  agg = (p_ref[0] + p_ref[1]) / cnt
  o_ref[...] = jnp.maximum(
      jnp.dot(agg, wl_ref[...], preferred_element_type=jnp.float32)
      + b_ref[...]
      + jnp.dot(x_ref[...], wr_ref[...], preferred_element_type=jnp.float32),
      0.0)


_R = 1000


def _tc_layer(p, cnt128, x, wlT, b, wrT):
  return pl.pallas_call(
      _tc_body,
      grid=(N // _R,),
      in_specs=[
          pl.BlockSpec((NC, _R, D), lambda i: (0, i, 0)),
          pl.BlockSpec((NC, _R, D), lambda i: (0, i, 0)),
          pl.BlockSpec((_R, D), lambda i: (i, 0)),
          pl.BlockSpec((D, D), lambda i: (0, 0)),
          pl.BlockSpec((1, D), lambda i: (0, 0)),
          pl.BlockSpec((D, D), lambda i: (0, 0)),
      ],
      out_specs=pl.BlockSpec((_R, D), lambda i: (i, 0)),
      out_shape=jax.ShapeDtypeStruct((N, D), jnp.float32),
  )(p, cnt128, x, wlT, b.reshape(1, D), wrT)


def kernel(x, edge_index, W1l, b1, W1r, W2l, b2, W2r):
  src = edge_index[0].astype(jnp.int32)
  dst = edge_index[1].astype(jnp.int32)
  pad = E_PAD - E
  pad_dst = N + (jnp.arange(pad, dtype=jnp.int32) % (N_TAB - N))
  src2 = jnp.concatenate([src, jnp.zeros((pad,), jnp.int32)]).reshape(-1, CHUNK)
  dst2 = jnp.concatenate([dst, pad_dst]).reshape(-1, CHUNK)

  znd = jnp.zeros((N_TAB, D), jnp.float32)
  ones = jnp.ones((CHUNK, D), jnp.float32)

  cnt128 = _sc_cnt(ones, src2, dst2, znd)
  p1 = _sc_agg(x, src2, dst2, znd)
  h = _tc_layer(p1, cnt128, x, W1l.T, b1, W1r.T)
  p2 = _sc_agg(h, src2, dst2, znd)
  out = _tc_layer(p2, cnt128, h, W2l.T, b2, W2r.T)
  return out


# speedup vs baseline: 2.9038x; 2.7872x over previous
import jax
import jax.numpy as jnp
from jax import lax
from jax.experimental import pallas as pl
from jax.experimental.pallas import tpu as pltpu
from jax.experimental.pallas import tpu_sc as plsc

N = 10000
E = 320000
D = 128

NC = 2
NS = 16
CHUNK = 128
NCHUNK = 2560
E_PAD = NCHUNK * CHUNK
CPW = NCHUNK // (NC * NS)
N_TAB = 10112
ZROWS = N_TAB // NS
GRP = 8

_MESH = plsc.VectorSubcoreMesh(
    core_axis_name="c", subcore_axis_name="s", num_cores=NC, num_subcores=NS)


def _sc_builder(kind):
  out_type = jax.ShapeDtypeStruct((NC, N_TAB, D), jnp.float32)
  scratch = [
      pltpu.VMEM((GRP, CHUNK), jnp.int32),
      pltpu.VMEM((GRP, CHUNK), jnp.int32),
      pltpu.VMEM((CHUNK, D), jnp.float32),
      pltpu.VMEM((CHUNK, D), jnp.float32),
      pltpu.VMEM_SHARED((N_TAB, D), jnp.float32),
      pltpu.SemaphoreType.DMA,
      pltpu.SemaphoreType.DMA,
  ]

  def body(table_hbm, src_hbm, dst_hbm, znd_hbm, p_hbm,
           srcv, dstv, rows_a, rows_b, agg_s, sem_a, sem_b):
    c = lax.axis_index("c")
    s = lax.axis_index("s")
    bufs = [rows_a, rows_b]
    sems = [sem_a, sem_b]

    pltpu.sync_copy(znd_hbm.at[pl.ds(s * ZROWS, ZROWS)],
                    agg_s.at[pl.ds(s * ZROWS, ZROWS)])
    if kind == "cnt":
      pltpu.sync_copy(table_hbm, rows_a)
    plsc.subcore_barrier()

    start = (c * NS + s) * CPW

    def group(g, carry):
      base = start + g * GRP
      if kind == "agg":
        pltpu.sync_copy(src_hbm.at[pl.ds(base, GRP)], srcv)
      pltpu.sync_copy(dst_hbm.at[pl.ds(base, GRP)], dstv)

      if kind == "agg":
        descs = [None, None]
        descs[0] = pltpu.async_copy(table_hbm.at[srcv.at[0]], bufs[0], sems[0])
        for j in range(GRP):
          if j + 1 < GRP:
            descs[(j + 1) % 2] = pltpu.async_copy(
                table_hbm.at[srcv.at[j + 1]], bufs[(j + 1) % 2],
                sems[(j + 1) % 2])
          descs[j % 2].wait()
          pltpu.sync_copy(bufs[j % 2], agg_s.at[dstv.at[j]], add=True)
      else:
        for j in range(GRP):
          pltpu.sync_copy(rows_a, agg_s.at[dstv.at[j]], add=True)
      return carry

    lax.fori_loop(0, CPW // GRP, group, 0)
    plsc.subcore_barrier()
    pltpu.sync_copy(agg_s.at[pl.ds(s * ZROWS, ZROWS)],
                    p_hbm.at[c, pl.ds(s * ZROWS, ZROWS)])

  return pl.kernel(body, out_type=out_type, mesh=_MESH, scratch_types=scratch)


_sc_agg = _sc_builder("agg")
_sc_cnt = _sc_builder("cnt")


def _tc_body(p_ref, c_ref, x_ref, wl_ref, b_ref, wr_ref, o_ref):
  cnt = jnp.maximum(c_ref[0, :, 0:1] + c_ref[1, :, 0:1], 1.0)
  agg = (p_ref[0] + p_ref[1]) / cnt
  o_ref[...] = jnp.maximum(
      jnp.dot(agg, wl_ref[...], preferred_element_type=jnp.float32)
      + b_ref[...]
      + jnp.dot(x_ref[...], wr_ref[...], preferred_element_type=jnp.float32),
      0.0)


_R = 1000


def _tc_layer(p, cnt128, x, wlT, b, wrT):
  return pl.pallas_call(
      _tc_body,
      grid=(N // _R,),
      in_specs=[
          pl.BlockSpec((NC, _R, D), lambda i: (0, i, 0)),
          pl.BlockSpec((NC, _R, D), lambda i: (0, i, 0)),
          pl.BlockSpec((_R, D), lambda i: (i, 0)),
          pl.BlockSpec((D, D), lambda i: (0, 0)),
          pl.BlockSpec((1, D), lambda i: (0, 0)),
          pl.BlockSpec((D, D), lambda i: (0, 0)),
      ],
      out_specs=pl.BlockSpec((_R, D), lambda i: (i, 0)),
      out_shape=jax.ShapeDtypeStruct((N, D), jnp.float32),
  )(p, cnt128, x, wlT, b.reshape(1, D), wrT)


def kernel(x, edge_index, W1l, b1, W1r, W2l, b2, W2r):
  src = edge_index[0].astype(jnp.int32)
  dst = edge_index[1].astype(jnp.int32)
  pad = E_PAD - E
  pad_idx = jnp.arange(pad, dtype=jnp.int32)
  pad_dst = N + (pad_idx % (N_TAB - N))
  pad_src = pad_idx % N
  src2 = jnp.concatenate([src, pad_src]).reshape(-1, CHUNK)
  dst2 = jnp.concatenate([dst, pad_dst]).reshape(-1, CHUNK)

  znd = jnp.zeros((N_TAB, D), jnp.float32)
  ones = jnp.ones((CHUNK, D), jnp.float32)

  cnt128 = _sc_cnt(ones, src2, dst2, znd)
  p1 = _sc_agg(x, src2, dst2, znd)
  h = _tc_layer(p1, cnt128, x, W1l.T, b1, W1r.T)
  p2 = _sc_agg(h, src2, dst2, znd)
  out = _tc_layer(p2, cnt128, h, W2l.T, b2, W2r.T)
  return out


# speedup vs baseline: 3.1047x; 1.0692x over previous
import jax
import jax.numpy as jnp
from jax import lax
from jax.experimental import pallas as pl
from jax.experimental.pallas import tpu as pltpu
from jax.experimental.pallas import tpu_sc as plsc

N = 10000
E = 320000
D = 128

NC = 2
NS = 16
CHUNK = 128
NCHUNK = 2560
E_PAD = NCHUNK * CHUNK
CPW = NCHUNK // (NC * NS)
N_TAB = 10112
ZROWS = N_TAB // NS
GRP = 16

_MESH = plsc.VectorSubcoreMesh(
    core_axis_name="c", subcore_axis_name="s", num_cores=NC, num_subcores=NS)


def _sc_builder(kind):
  out_type = jax.ShapeDtypeStruct((NC, N_TAB, D), jnp.float32)
  scratch = [
      pltpu.VMEM((GRP, CHUNK), jnp.int32),
      pltpu.VMEM((GRP, CHUNK), jnp.int32),
      pltpu.VMEM((CHUNK, D), jnp.float32),
      pltpu.VMEM((CHUNK, D), jnp.float32),
      pltpu.VMEM_SHARED((N_TAB, D), jnp.float32),
      pltpu.SemaphoreType.DMA,
      pltpu.SemaphoreType.DMA,
  ]

  def body(table_hbm, src_hbm, dst_hbm, znd_hbm, p_hbm,
           srcv, dstv, rows_a, rows_b, agg_s, sem_a, sem_b):
    c = lax.axis_index("c")
    s = lax.axis_index("s")
    bufs = [rows_a, rows_b]
    sems = [sem_a, sem_b]

    pltpu.sync_copy(znd_hbm.at[pl.ds(s * ZROWS, ZROWS)],
                    agg_s.at[pl.ds(s * ZROWS, ZROWS)])
    if kind == "cnt":
      pltpu.sync_copy(table_hbm, rows_a)
    plsc.subcore_barrier()

    start = (c * NS + s) * CPW

    def group(g, carry):
      base = start + g * GRP
      if kind == "agg":
        pltpu.sync_copy(src_hbm.at[pl.ds(base, GRP)], srcv)
      pltpu.sync_copy(dst_hbm.at[pl.ds(base, GRP)], dstv)

      if kind == "agg":
        descs = [None, None]
        descs[0] = pltpu.async_copy(table_hbm.at[srcv.at[0]], bufs[0], sems[0])
        for j in range(GRP):
          if j + 1 < GRP:
            descs[(j + 1) % 2] = pltpu.async_copy(
                table_hbm.at[srcv.at[j + 1]], bufs[(j + 1) % 2],
                sems[(j + 1) % 2])
          descs[j % 2].wait()
          pltpu.sync_copy(bufs[j % 2], agg_s.at[dstv.at[j]], add=True)
      else:
        for j in range(GRP):
          pltpu.sync_copy(rows_a, agg_s.at[dstv.at[j]], add=True)
      return carry

    lax.fori_loop(0, CPW // GRP, group, 0)
    plsc.subcore_barrier()
    pltpu.sync_copy(agg_s.at[pl.ds(s * ZROWS, ZROWS)],
                    p_hbm.at[c, pl.ds(s * ZROWS, ZROWS)])

  return pl.kernel(body, out_type=out_type, mesh=_MESH, scratch_types=scratch)


_sc_agg = _sc_builder("agg")
_sc_cnt = _sc_builder("cnt")


def _tc_body(p_ref, c_ref, x_ref, wl_ref, b_ref, wr_ref, o_ref):
  cnt = jnp.maximum(c_ref[0, :, 0:1] + c_ref[1, :, 0:1], 1.0)
  agg = (p_ref[0] + p_ref[1]) / cnt
  o_ref[...] = jnp.maximum(
      jnp.dot(agg, wl_ref[...], preferred_element_type=jnp.float32)
      + b_ref[...]
      + jnp.dot(x_ref[...], wr_ref[...], preferred_element_type=jnp.float32),
      0.0)


_R = 1000


def _tc_layer(p, cnt128, x, wlT, b, wrT):
  return pl.pallas_call(
      _tc_body,
      grid=(N // _R,),
      in_specs=[
          pl.BlockSpec((NC, _R, D), lambda i: (0, i, 0)),
          pl.BlockSpec((NC, _R, D), lambda i: (0, i, 0)),
          pl.BlockSpec((_R, D), lambda i: (i, 0)),
          pl.BlockSpec((D, D), lambda i: (0, 0)),
          pl.BlockSpec((1, D), lambda i: (0, 0)),
          pl.BlockSpec((D, D), lambda i: (0, 0)),
      ],
      out_specs=pl.BlockSpec((_R, D), lambda i: (i, 0)),
      out_shape=jax.ShapeDtypeStruct((N, D), jnp.float32),
  )(p, cnt128, x, wlT, b.reshape(1, D), wrT)


def kernel(x, edge_index, W1l, b1, W1r, W2l, b2, W2r):
  src = edge_index[0].astype(jnp.int32)
  dst = edge_index[1].astype(jnp.int32)
  pad = E_PAD - E
  pad_idx = jnp.arange(pad, dtype=jnp.int32)
  pad_dst = N + (pad_idx % (N_TAB - N))
  pad_src = pad_idx % N
  src2 = jnp.concatenate([src, pad_src]).reshape(-1, CHUNK)
  dst2 = jnp.concatenate([dst, pad_dst]).reshape(-1, CHUNK)

  znd = jnp.zeros((N_TAB, D), jnp.float32)
  ones = jnp.ones((CHUNK, D), jnp.float32)

  cnt128 = _sc_cnt(ones, src2, dst2, znd)
  p1 = _sc_agg(x, src2, dst2, znd)
  h = _tc_layer(p1, cnt128, x, W1l.T, b1, W1r.T)
  p2 = _sc_agg(h, src2, dst2, znd)
  out = _tc_layer(p2, cnt128, h, W2l.T, b2, W2r.T)
  return out
